# Initial kernel scaffold; baseline (speedup 1.0000x reference)
import jax
import jax.numpy as jnp
from jax.experimental import pallas as pl


def kernel(x, edge_index, edge_attr, params):
    raise NotImplementedError("write your pallas kernel here")


# speedup vs baseline: 1.8388x; 1.8388x over previous
import functools

import jax
import jax.numpy as jnp
from jax import lax
from jax.experimental import pallas as pl
from jax.experimental.pallas import tpu as pltpu
from jax.experimental.pallas import tpu_sc as plsc

_NC = 2
_NS = 16
_NW = _NC * _NS
_SUB = 128
_K = 16
_GROUP = _SUB * _K


@functools.lru_cache(maxsize=None)
def _make_sc_gather(n_rows, d_pad, e_pad):
    rows_per_w = e_pad // _NW
    n_groups = rows_per_w // _GROUP
    mesh = plsc.VectorSubcoreMesh(core_axis_name="c", subcore_axis_name="s")

    @functools.partial(
        pl.kernel,
        mesh=mesh,
        out_type=jax.ShapeDtypeStruct((e_pad, d_pad), jnp.float32),
        compiler_params=pltpu.CompilerParams(use_tc_tiling_on_sc=False),
        scratch_types=[
            pltpu.VMEM((_GROUP,), jnp.int32),
            pltpu.VMEM((_GROUP, d_pad), jnp.float32),
            pltpu.SemaphoreType.DMA,
        ],
    )
    def gather_kernel(table_hbm, idx_hbm, out_hbm, idx_v, rows_v, sem):
        wid = lax.axis_index("s") * _NC + lax.axis_index("c")
        base = wid * rows_per_w

        def body(g, carry):
            off = base + g * _GROUP
            pltpu.sync_copy(idx_hbm.at[pl.ds(off, _GROUP)], idx_v)
            copies = []
            for j in range(_K):
                copies.append(
                    pltpu.async_copy(
                        table_hbm.at[idx_v.at[pl.ds(j * _SUB, _SUB)]],
                        rows_v.at[pl.ds(j * _SUB, _SUB)],
                        sem,
                    )
                )
            for cp in copies:
                cp.wait()
            pltpu.sync_copy(rows_v, out_hbm.at[pl.ds(off, _GROUP)])
            return carry

        lax.fori_loop(0, n_groups, body, 0)

    return gather_kernel


def _sc_gather_rows(table, idx):
    n, d = table.shape
    e = idx.shape[0]
    unit = _NW * _GROUP
    e_pad = ((e + unit - 1) // unit) * unit
    if e_pad != e:
        idx = jnp.concatenate([idx, jnp.zeros((e_pad - e,), jnp.int32)])
    out = _make_sc_gather(n, d, e_pad)(table, idx)
    return out[:e]


def _pad_cols(a, d_pad):
    return jnp.pad(a, ((0, 0), (0, d_pad - a.shape[1])))


def _leaky(v, s=0.01):
    return jnp.where(v >= 0, v, s * v)


def _elu(v):
    return jnp.where(v > 0, v, jnp.expm1(v))


def _gru(xv, h, Wih, Whh, bih, bhh):
    gi = xv @ Wih.T + bih
    gh = h @ Whh.T + bhh
    ir, iz, inn = jnp.split(gi, 3, axis=-1)
    hr, hz, hn = jnp.split(gh, 3, axis=-1)
    r = jax.nn.sigmoid(ir + hr)
    z = jax.nn.sigmoid(iz + hz)
    g = jnp.tanh(inn + r * hn)
    return (1.0 - z) * g + z * h


def _edge_softmax(a, dst, n):
    e = jnp.exp(a)
    s = jax.ops.segment_sum(e, dst, num_segments=n)
    return e / (s[dst] + 1e-16)


def kernel(x, edge_index, edge_attr, params):
    p = params
    src = edge_index[0]
    dst = edge_index[1]
    N = x.shape[0]
    H = p["lin1_W"].shape[0]

    x = _leaky(x @ p["lin1_W"].T + p["lin1_b"])

    xsrc = _sc_gather_rows(_pad_cols(x, 48), src)[:, :H]
    xj = _leaky(jnp.concatenate([xsrc, edge_attr], axis=-1) @ p["gate_lin1_W"].T)
    a = _leaky((xj @ p["gate_att_l"]) + (x @ p["gate_att_r"])[dst])
    alpha = _edge_softmax(a, dst, N)
    h = jax.ops.segment_sum(
        (xj @ p["gate_lin2_W"].T) * alpha[:, None], dst, num_segments=N
    ) + p["gate_bias"]
    x = jax.nn.relu(
        _gru(_elu(h), x, p["gru_Wih"][0], p["gru_Whh"][0], p["gru_bih"][0], p["gru_bhh"][0])
    )

    nl = p["gru_Wih"].shape[0]
    for l in range(nl - 1):
        xt = x @ p["conv_W"][l].T
        xts = _sc_gather_rows(_pad_cols(xt, 48), src)[:, :H]
        a = _leaky((xts @ p["conv_att_src"][l]) + (xt @ p["conv_att_dst"][l])[dst])
        alpha = _edge_softmax(a, dst, N)
        h = jax.ops.segment_sum(xts * alpha[:, None], dst, num_segments=N) + p["conv_bias"][l]
        x = jax.nn.relu(
            _gru(_elu(h), x, p["gru_Wih"][l + 1], p["gru_Whh"][l + 1],
                 p["gru_bih"][l + 1], p["gru_bhh"][l + 1])
        )

    out = jax.nn.relu(jnp.sum(x, axis=0, keepdims=True))
    for _t in range(3):
        xs = x @ p["mol_Wsrc"].T
        od = out @ p["mol_Wdst"].T
        a = _leaky((xs @ p["mol_att_src"]) + (od @ p["mol_att_dst"])[0])
        m = jnp.max(a)
        m = jnp.where(jnp.isfinite(m), m, 0.0)
        e = jnp.exp(a - m)
        s = jnp.sum(e)
        alpha = e / (s + 1e-16)
        h = jnp.sum(xs * alpha[:, None], axis=0, keepdims=True) + p["mol_bias"]
        out = jax.nn.relu(
            _gru(_elu(h), out, p["mol_gru_Wih"], p["mol_gru_Whh"],
                 p["mol_gru_bih"], p["mol_gru_bhh"])
        )
    return out @ p["lin2_W"].T + p["lin2_b"]
